# baseline (speedup 1.0000x reference)
import functools

import jax
import jax.numpy as jnp
from jax import lax
from jax.experimental import pallas as pl
from jax.experimental.pallas import tpu as pltpu
from jax.experimental.pallas import tpu_sc as plsc

_NW = 32
_NB = 8
_NF = 5


@functools.lru_cache(maxsize=None)
def _make_gather(b: int, l: int, vocab: int, embed: int):
    bpw = b // _NW
    l_splits = []
    off = 0
    while off < l:
        n = min(128, l - off)
        l_splits.append((off, n))
        off += n
    assert all(o % 8 == 0 for o, _ in l_splits)
    assert b % _NW == 0 and bpw % _NB == 0

    mesh = plsc.VectorSubcoreMesh(core_axis_name="c", subcore_axis_name="s")

    @functools.partial(
        pl.kernel,
        out_type=jax.ShapeDtypeStruct((b, l, embed), jnp.float32),
        mesh=mesh,
        scratch_types=[
            pltpu.VMEM((bpw, l), jnp.int32),
            pltpu.VMEM((_NB, l, embed), jnp.float32),
        ]
        + [pltpu.SemaphoreType.DMA] * (2 * _NB),
        compiler_params=pltpu.CompilerParams(use_tc_tiling_on_sc=False),
    )
    def grab(table_hbm, idx_hbm, out_hbm, idx_v, rows_v, *sems):
        g_sem = sems[:_NB]
        o_sem = sems[_NB:]
        wid = lax.axis_index("s") * 2 + lax.axis_index("c")
        base = wid * bpw
        pltpu.sync_copy(idx_hbm.at[pl.ds(base, bpw)], idx_v)

        def start_gather(bi, slot):
            for off, n in l_splits:
                pltpu.async_copy(
                    table_hbm.at[idx_v.at[bi, pl.ds(off, n)]],
                    rows_v.at[slot, pl.ds(off, n)],
                    g_sem[slot],
                )

        def wait_gather(slot):
            pltpu.make_async_copy(
                table_hbm.at[idx_v.at[0]], rows_v.at[slot], g_sem[slot]
            ).wait()

        for s in range(_NF):
            start_gather(s, s)

        def body(grp, _):
            for s in range(_NB):
                bi = grp * _NB + s
                t = (s + _NF) % _NB

                @pl.when(bi + _NF < bpw)
                def _():
                    @pl.when(bi + _NF >= _NB)
                    def _():
                        pltpu.make_async_copy(
                            rows_v.at[t], out_hbm.at[base], o_sem[t]
                        ).wait()

                    start_gather(bi + _NF, t)

                wait_gather(s)
                pltpu.async_copy(
                    rows_v.at[s], out_hbm.at[base + bi], o_sem[s]
                )
            return 0

        lax.fori_loop(0, bpw // _NB, body, 0)

        for s in range(_NB):
            pltpu.make_async_copy(
                rows_v.at[s], out_hbm.at[base], o_sem[s]
            ).wait()

    return grab


def kernel(input, voc_emb_weight):
    b, l = input.shape
    vocab, embed = voc_emb_weight.shape
    grab = _make_gather(b, l, vocab, embed)
    return grab(voc_emb_weight, input.astype(jnp.int32))


# speedup vs baseline: 1.0033x; 1.0033x over previous
import functools

import jax
import jax.numpy as jnp
from jax import lax
from jax.experimental import pallas as pl
from jax.experimental.pallas import tpu as pltpu
from jax.experimental.pallas import tpu_sc as plsc

_NW = 32
_NB = 8
_NF = 5


@functools.lru_cache(maxsize=None)
def _make_gather(b: int, l: int, vocab: int, embed: int):
    bpw = b // _NW
    l_splits = []
    off = 0
    while off < l:
        n = min(128, l - off)
        l_splits.append((off, n))
        off += n
    assert all(o % 8 == 0 for o, _ in l_splits)
    assert b % _NW == 0 and bpw % _NB == 0

    mesh = plsc.VectorSubcoreMesh(core_axis_name="c", subcore_axis_name="s")

    @functools.partial(
        pl.kernel,
        out_type=jax.ShapeDtypeStruct((b, l, embed), jnp.float32),
        mesh=mesh,
        scratch_types=[
            pltpu.VMEM((bpw, l), jnp.int32),
            pltpu.VMEM((_NB, l, embed), jnp.float32),
        ]
        + [pltpu.SemaphoreType.DMA] * (2 * _NB),
        compiler_params=pltpu.CompilerParams(
            use_tc_tiling_on_sc=False, needs_layout_passes=False
        ),
    )
    def grab(table_hbm, idx_hbm, out_hbm, idx_v, rows_v, *sems):
        g_sem = sems[:_NB]
        o_sem = sems[_NB:]
        wid = lax.axis_index("s") * 2 + lax.axis_index("c")
        base = wid * bpw
        pltpu.sync_copy(idx_hbm.at[pl.ds(base, bpw)], idx_v)

        def start_gather(bi, slot):
            for off, n in l_splits:
                pltpu.async_copy(
                    table_hbm.at[idx_v.at[bi, pl.ds(off, n)]],
                    rows_v.at[slot, pl.ds(off, n)],
                    g_sem[slot],
                )

        def wait_gather(slot):
            pltpu.make_async_copy(
                table_hbm.at[idx_v.at[0]], rows_v.at[slot], g_sem[slot]
            ).wait()

        for s in range(_NF):
            start_gather(s, s)

        def body(grp, _):
            for s in range(_NB):
                bi = grp * _NB + s
                t = (s + _NF) % _NB

                @pl.when(bi + _NF < bpw)
                def _():
                    @pl.when(bi + _NF >= _NB)
                    def _():
                        pltpu.make_async_copy(
                            rows_v.at[t], out_hbm.at[base], o_sem[t]
                        ).wait()

                    start_gather(bi + _NF, t)

                wait_gather(s)
                pltpu.async_copy(
                    rows_v.at[s], out_hbm.at[base + bi], o_sem[s]
                )
            return 0

        lax.fori_loop(0, bpw // _NB, body, 0)

        for s in range(_NB):
            pltpu.make_async_copy(
                rows_v.at[s], out_hbm.at[base], o_sem[s]
            ).wait()

    return grab


def kernel(input, voc_emb_weight):
    b, l = input.shape
    vocab, embed = voc_emb_weight.shape
    grab = _make_gather(b, l, vocab, embed)
    return grab(voc_emb_weight, input.astype(jnp.int32))
